# baseline (speedup 1.0000x reference)
import functools
import jax
import jax.numpy as jnp
from jax import lax
from jax.experimental import pallas as pl
from jax.experimental.pallas import tpu as pltpu
from jax.experimental.pallas import tpu_sc as plsc

_CH = 8


def kernel(x):
    B, C, D = x.shape
    info = plsc.get_sparse_core_info()
    NC, NS = info.num_cores, info.num_subcores
    NW = NC * NS
    rows_per_w = B // NW
    n_chunks = rows_per_w // _CH
    nd = D // 16

    mesh = plsc.VectorSubcoreMesh(core_axis_name="c", subcore_axis_name="s")

    _gdn = lax.GatherDimensionNumbers(
        offset_dims=(), collapsed_slice_dims=(0,), start_index_map=(0,))

    def _perm(v, idx):
        return lax.gather(v, idx[:, None], _gdn, slice_sizes=(1,),
                          mode=lax.GatherScatterMode.PROMISE_IN_BOUNDS)

    def _bfly(v, op, iot):
        for d in (1, 2, 4, 8):
            v = op(v, _perm(v, iot ^ d))
        return v

    @functools.partial(
        pl.kernel,
        mesh=mesh,
        out_type=[
            jax.ShapeDtypeStruct((B, D), jnp.float32),
            jax.ShapeDtypeStruct((B, C, D), jnp.float32),
        ],
        scratch_types=[
            pltpu.VMEM((_CH, C, D), jnp.float32),
            pltpu.VMEM((_CH, C, D), jnp.float32),
            pltpu.VMEM((_CH, D), jnp.float32),
        ],
    )
    def run(x_hbm, logits_hbm, latent_hbm, x_buf, out_buf, log_buf):
        wid = lax.axis_index("s") * NC + lax.axis_index("c")
        w_base = wid * rows_per_w
        iot = lax.broadcasted_iota(jnp.int32, (16,), 0)

        def chunk_body(ch, carry):
            base = w_base + ch * _CH
            pltpu.sync_copy(x_hbm.at[pl.ds(base, _CH)], x_buf)

            def row_body(r, carry2):
                ss = []
                for k in range(nd):
                    acc = x_buf[r, 0, pl.ds(k * 16, 16)]
                    acc = acc * acc
                    for c in range(1, C):
                        v = x_buf[r, c, pl.ds(k * 16, 16)]
                        acc = acc + v * v
                    ss.append(acc)
                for k in range(nd):
                    s = ss[k]
                    i = lax.bitcast_convert_type(s, jnp.int32)
                    y = lax.bitcast_convert_type(
                        jnp.int32(0x5F3759DF) - (i >> 1), jnp.float32)
                    for _ in range(3):
                        y = y * (1.5 - 0.5 * s * y * y)
                    log_buf[r, pl.ds(k * 16, 16)] = jnp.where(
                        s > 0.0, s * y, 0.0)
                m = ss[0]
                for k in range(1, nd):
                    m = jnp.maximum(m, ss[k])
                gm = _bfly(m, jnp.maximum, iot)
                cand = jnp.where(ss[0] == gm, iot, D)
                for k in range(1, nd):
                    cand = jnp.minimum(
                        cand, jnp.where(ss[k] == gm, iot + 16 * k, D))
                gi = _bfly(cand, jnp.minimum, iot)
                masks = [
                    jnp.where((iot + 16 * k) == gi, 1.0, 0.0).astype(jnp.float32)
                    for k in range(nd)
                ]
                for c in range(C):
                    for k in range(nd):
                        out_buf[r, c, pl.ds(k * 16, 16)] = (
                            x_buf[r, c, pl.ds(k * 16, 16)] * masks[k])
                return carry2

            lax.fori_loop(0, _CH, row_body, 0)
            pltpu.sync_copy(out_buf, latent_hbm.at[pl.ds(base, _CH)])
            pltpu.sync_copy(log_buf, logits_hbm.at[pl.ds(base, _CH)])
            return carry

        lax.fori_loop(0, n_chunks, chunk_body, 0)

    logits, latent = run(x)
    return (logits, latent.reshape(B, C * D))
